# baseline (speedup 1.0000x reference)
import jax
import jax.numpy as jnp
from jax.experimental import pallas as pl
from jax.experimental.pallas import tpu as pltpu

_MT = 16
_MY = 32
_BR = 2048


def _body(xa_ref, xb_ref, t_ref, w_ref, b_ref, s_ref, o_ref):
    dh = xa_ref.shape[1]
    wb = w_ref[...].astype(jnp.bfloat16)
    xa = xa_ref[...].astype(jnp.bfloat16)
    xc = xb_ref[...].astype(jnp.bfloat16)
    logits = jax.lax.dot_general(
        xa, wb[:, :dh], (((1,), (1,)), ((), ())),
        preferred_element_type=jnp.float32)
    logits = logits + jax.lax.dot_general(
        xc, wb[:, dh:], (((1,), (1,)), ((), ())),
        preferred_element_type=jnp.float32)
    tb = t_ref[...]
    gid = jax.lax.broadcasted_iota(jnp.int32, logits.shape, 1) // _MY
    masked = jnp.where(gid == tb, logits, 0.0).astype(jnp.bfloat16)
    acc = jnp.dot(masked, s_ref[...], preferred_element_type=jnp.float32)
    e = jax.lax.broadcasted_iota(jnp.int32, (tb.shape[0], _MT), 1)
    onehot = (e == tb).astype(jnp.float32)
    acc = acc + jnp.dot(onehot, b_ref[...], preferred_element_type=jnp.float32)
    m = jnp.max(acc, axis=1, keepdims=True)
    p = jnp.exp(acc - m)
    o_ref[...] = p / jnp.sum(p, axis=1, keepdims=True)


def kernel(x, t, W, b):
    n, d = x.shape
    mt, my, _ = W.shape
    wr = W.reshape(mt * my, d)
    t2 = t.reshape(n, 1)
    sel = jnp.tile(jnp.eye(my, dtype=jnp.bfloat16), (mt, 1))
    dh = d // 2
    grid = (n // _BR,)
    return pl.pallas_call(
        _body,
        grid=grid,
        in_specs=[
            pl.BlockSpec((_BR, dh), lambda i: (i, 0)),
            pl.BlockSpec((_BR, dh), lambda i: (i, 1)),
            pl.BlockSpec((_BR, 1), lambda i: (i, 0)),
            pl.BlockSpec((mt * my, d), lambda i: (0, 0)),
            pl.BlockSpec((mt, my), lambda i: (0, 0)),
            pl.BlockSpec((mt * my, my), lambda i: (0, 0)),
        ],
        out_specs=pl.BlockSpec((_BR, my), lambda i: (i, 0)),
        out_shape=jax.ShapeDtypeStruct((n, my), x.dtype),
        compiler_params=pltpu.CompilerParams(
            dimension_semantics=("parallel",)),
    )(x, x, t2, wr, b, sel)
